# Initial kernel scaffold; baseline (speedup 1.0000x reference)
import jax
import jax.numpy as jnp
from jax.experimental import pallas as pl


def kernel(p, x, o, W, gamma, beta):
    raise NotImplementedError("write your pallas kernel here")


# speedup vs baseline: 12.1167x; 12.1167x over previous
import functools

import jax
import jax.numpy as jnp
from jax import lax
from jax.experimental import pallas as pl
from jax.experimental.pallas import tpu as pltpu
from jax.experimental.pallas import tpu_sc as plsc

B = 4
N_PER = 4096
STRIDE = 4
NSAMPLE = 16
IN_C = 256
OUT_C = 512
M_PER = N_PER // STRIDE
EPS = 1e-5
KPAD = 384


def _fps_body(pt_ref, cx_ref, cy_ref, cz_ref):
    px = pt_ref[0]
    py = pt_ref[1]
    pz = pt_ref[2]

    lane_n = lax.broadcasted_iota(jnp.int32, (B, N_PER), 1)
    lane_m = lax.broadcasted_iota(jnp.int32, (B, M_PER), 1)

    c0x = px[:, 0:1]
    c0y = py[:, 0:1]
    c0z = pz[:, 0:1]
    dx = px - c0x
    dy = py - c0y
    dz = pz - c0z
    dist = (dx * dx + dy * dy) + dz * dz

    accx = jnp.where(lane_m == 0, c0x, 0.0)
    accy = jnp.where(lane_m == 0, c0y, 0.0)
    accz = jnp.where(lane_m == 0, c0z, 0.0)

    def body(i, state):
        dist, accx, accy, accz = state
        mx = jnp.max(dist, axis=1, keepdims=True)
        nxt = jnp.min(jnp.where(dist == mx, lane_n, N_PER), axis=1,
                      keepdims=True)
        onehot = lane_n == nxt
        cx = jnp.sum(jnp.where(onehot, px, 0.0), axis=1, keepdims=True)
        cy = jnp.sum(jnp.where(onehot, py, 0.0), axis=1, keepdims=True)
        cz = jnp.sum(jnp.where(onehot, pz, 0.0), axis=1, keepdims=True)
        dx = px - cx
        dy = py - cy
        dz = pz - cz
        d = (dx * dx + dy * dy) + dz * dz
        dist = jnp.minimum(dist, d)
        accx = jnp.where(lane_m == i, cx, accx)
        accy = jnp.where(lane_m == i, cy, accy)
        accz = jnp.where(lane_m == i, cz, accz)
        return dist, accx, accy, accz

    _, accx, accy, accz = lax.fori_loop(
        1, M_PER, body, (dist, accx, accy, accz))
    cx_ref[...] = accx
    cy_ref[...] = accy
    cz_ref[...] = accz


def _run_fps(pt):
    out = jax.ShapeDtypeStruct((B, M_PER), jnp.float32)
    return pl.pallas_call(
        _fps_body,
        out_shape=(out, out, out),
    )(pt)


_KNN_MB = 128


def _knn_body(cx_ref, cy_ref, cz_ref, pt_ref, knn_ref):
    b = pl.program_id(0)
    px = pt_ref[0, 0:1, :]
    py = pt_ref[0, 1:2, :]
    pz = pt_ref[0, 2:3, :]
    cx = cx_ref[0]
    cy = cy_ref[0]
    cz = cz_ref[0]

    dx = cx - px
    dy = cy - py
    dz = cz - pz
    d = (dx * dx + dy * dy) + dz * dz

    lane_n = lax.broadcasted_iota(jnp.int32, (_KNN_MB, N_PER), 1)
    base = b * N_PER
    inf = jnp.float32(jnp.inf)
    for k in range(NSAMPLE):
        mn = jnp.min(d, axis=1, keepdims=True)
        am = jnp.min(jnp.where(d == mn, lane_n, N_PER), axis=1,
                     keepdims=True)
        knn_ref[0, :, k:k + 1] = am + base
        d = jnp.where(lane_n == am, inf, d)


def _run_knn(cxT, cyT, czT, ptb):
    grid = (B, M_PER // _KNN_MB)
    cspec = pl.BlockSpec((1, _KNN_MB, 1), lambda b, m: (b, m, 0))
    return pl.pallas_call(
        _knn_body,
        grid=grid,
        in_specs=[
            cspec, cspec, cspec,
            pl.BlockSpec((1, 3, N_PER), lambda b, m: (b, 0, 0)),
        ],
        out_specs=pl.BlockSpec((1, _KNN_MB, NSAMPLE), lambda b, m: (b, m, 0)),
        out_shape=jax.ShapeDtypeStruct((B, M_PER, NSAMPLE), jnp.int32),
        compiler_params=pltpu.CompilerParams(
            dimension_semantics=("parallel", "parallel")),
    )(cxT, cyT, czT, ptb)


_PROJ_RB = 1024


def _proj_body(f_ref, w_ref, y_ref):
    y_ref[...] = jnp.dot(f_ref[...], w_ref[...],
                         preferred_element_type=jnp.float32)


def _run_proj(feats, wt):
    n = feats.shape[0]
    grid = (n // _PROJ_RB,)
    return pl.pallas_call(
        _proj_body,
        grid=grid,
        in_specs=[
            pl.BlockSpec((_PROJ_RB, KPAD), lambda i: (i, 0)),
            pl.BlockSpec((KPAD, OUT_C), lambda i: (0, 0)),
        ],
        out_specs=pl.BlockSpec((_PROJ_RB, OUT_C), lambda i: (i, 0)),
        out_shape=jax.ShapeDtypeStruct((n, OUT_C), jnp.float32),
        compiler_params=pltpu.CompilerParams(
            dimension_semantics=("parallel",)),
    )(feats, wt)


_SC_CHUNK = 128


def _run_sc_gather(table, idx):
    total = idx.shape[0]
    info = plsc.get_sparse_core_info()
    nw = info.num_cores * info.num_subcores
    per_w = total // nw
    nchunk = per_w // _SC_CHUNK
    mesh = plsc.VectorSubcoreMesh(core_axis_name="c", subcore_axis_name="s")

    @functools.partial(
        pl.kernel,
        mesh=mesh,
        out_type=jax.ShapeDtypeStruct((total, OUT_C), jnp.float32),
        scratch_types=[
            pltpu.VMEM((_SC_CHUNK,), jnp.int32),
            pltpu.VMEM((_SC_CHUNK, OUT_C), jnp.float32),
            pltpu.SemaphoreType.DMA,
        ],
    )
    def gather_k(table_hbm, idx_hbm, out_hbm, idx_v, rows_v, sem):
        wid = lax.axis_index("s") * info.num_cores + lax.axis_index("c")
        base = wid * per_w

        def body(j, carry):
            off = base + j * _SC_CHUNK
            pltpu.sync_copy(idx_hbm.at[pl.ds(off, _SC_CHUNK)], idx_v)
            pltpu.async_copy(table_hbm.at[idx_v], rows_v, sem).wait()
            pltpu.sync_copy(rows_v, out_hbm.at[pl.ds(off, _SC_CHUNK)])
            return carry

        lax.fori_loop(0, nchunk, body, 0)

    return gather_k(table, idx)


_LN_MB = 128


def _ln_body(g_ref, cx_ref, cy_ref, cz_ref, w3_ref, gam_ref, bet_ref, o_ref):
    cx = cx_ref[0]
    cy = cy_ref[0]
    cz = cz_ref[0]
    wx = w3_ref[0:1, :]
    wy = w3_ref[1:2, :]
    wz = w3_ref[2:3, :]
    projc = cx * wx + cy * wy + cz * wz
    gam = gam_ref[0:1, :]
    bet = bet_ref[0:1, :]

    acc = jnp.full((_LN_MB, OUT_C), -jnp.inf, dtype=jnp.float32)
    for k in range(NSAMPLE):
        z = g_ref[0, :, k] - projc
        mu = jnp.mean(z, axis=1, keepdims=True)
        zc = z - mu
        var = jnp.mean(zc * zc, axis=1, keepdims=True)
        y = zc * lax.rsqrt(var + EPS) * gam + bet
        acc = jnp.maximum(acc, y)
    o_ref[...] = jnp.maximum(acc, 0.0)


def _run_ln(g4, cxT, cyT, czT, w3, gamma2, beta2):
    grid = (B, M_PER // _LN_MB)
    cspec = pl.BlockSpec((1, _LN_MB, 1), lambda b, m: (b, m, 0))
    vspec = pl.BlockSpec((1, OUT_C), lambda b, m: (0, 0))
    return pl.pallas_call(
        _ln_body,
        grid=grid,
        in_specs=[
            pl.BlockSpec((1, _LN_MB, NSAMPLE, OUT_C), lambda b, m: (b, m, 0, 0)),
            cspec, cspec, cspec,
            pl.BlockSpec((3, OUT_C), lambda b, m: (0, 0)),
            vspec, vspec,
        ],
        out_specs=pl.BlockSpec((_LN_MB, OUT_C), lambda b, m: (b * (M_PER // _LN_MB) + m, 0)),
        out_shape=jax.ShapeDtypeStruct((B * M_PER, OUT_C), jnp.float32),
        compiler_params=pltpu.CompilerParams(
            dimension_semantics=("parallel", "parallel")),
    )(g4, cxT, cyT, czT, w3, gamma2, beta2)


def kernel(p, x, o, W, gamma, beta):
    pb = p.reshape(B, N_PER, 3)
    pt = pb.transpose(2, 0, 1)
    ptb = pb.transpose(0, 2, 1)
    cx, cy, cz = _run_fps(pt)
    cxT, cyT, czT = cx[..., None], cy[..., None], cz[..., None]

    knn = _run_knn(cxT, cyT, czT, ptb)

    feats = jnp.concatenate([p, x], axis=1)
    feats = jnp.pad(feats, ((0, 0), (0, KPAD - feats.shape[1])))
    wt = jnp.pad(W, ((0, 0), (0, KPAD - W.shape[1]))).T
    y_all = _run_proj(feats, wt)

    idx_flat = knn.reshape(B * NSAMPLE * M_PER)
    grouped = _run_sc_gather(y_all, idx_flat)
    g4 = grouped.reshape(B, M_PER, NSAMPLE, OUT_C)

    w3 = W[:, :3].T
    x_out = _run_ln(g4, cxT, cyT, czT, w3, gamma[None, :], beta[None, :])

    n_p = jnp.stack([cx, cy, cz], axis=-1).reshape(B * M_PER, 3)
    n_o = (jnp.arange(1, B + 1) * M_PER).astype(jnp.int32)
    return (n_p, x_out, n_o)


# speedup vs baseline: 12.7908x; 1.0556x over previous
import functools

import jax
import jax.numpy as jnp
from jax import lax
from jax.experimental import pallas as pl
from jax.experimental.pallas import tpu as pltpu
from jax.experimental.pallas import tpu_sc as plsc

B = 4
N_PER = 4096
STRIDE = 4
NSAMPLE = 16
IN_C = 256
OUT_C = 512
M_PER = N_PER // STRIDE
EPS = 1e-5
KPAD = 384


_FS = 8
_FL = N_PER // _FS
_MS = 8
_ML = M_PER // _MS


def _fps_body(pt_ref, cx_ref, cy_ref, cz_ref):
    px = pt_ref[0]
    py = pt_ref[1]
    pz = pt_ref[2]

    io = (lax.broadcasted_iota(jnp.int32, (B, _FS, _FL), 1) * _FL
          + lax.broadcasted_iota(jnp.int32, (B, _FS, _FL), 2))
    im = (lax.broadcasted_iota(jnp.int32, (B, _MS, _ML), 1) * _ML
          + lax.broadcasted_iota(jnp.int32, (B, _MS, _ML), 2))

    def _rmax(a):
        return jnp.max(jnp.max(a, axis=2, keepdims=True), axis=1,
                       keepdims=True)

    def _rmin(a):
        return jnp.min(jnp.min(a, axis=2, keepdims=True), axis=1,
                       keepdims=True)

    def _rsum(a):
        return jnp.sum(jnp.sum(a, axis=2, keepdims=True), axis=1,
                       keepdims=True)

    c0x = px[:, 0:1, 0:1]
    c0y = py[:, 0:1, 0:1]
    c0z = pz[:, 0:1, 0:1]
    dx = px - c0x
    dy = py - c0y
    dz = pz - c0z
    dist = (dx * dx + dy * dy) + dz * dz

    accx = jnp.where(im == 0, c0x, 0.0)
    accy = jnp.where(im == 0, c0y, 0.0)
    accz = jnp.where(im == 0, c0z, 0.0)

    def body(i, state):
        dist, accx, accy, accz = state
        mx = _rmax(dist)
        nxt = _rmin(jnp.where(dist == mx, io, N_PER))
        onehot = io == nxt
        cx = _rsum(jnp.where(onehot, px, 0.0))
        cy = _rsum(jnp.where(onehot, py, 0.0))
        cz = _rsum(jnp.where(onehot, pz, 0.0))
        dx = px - cx
        dy = py - cy
        dz = pz - cz
        d = (dx * dx + dy * dy) + dz * dz
        dist = jnp.minimum(dist, d)
        accx = jnp.where(im == i, cx, accx)
        accy = jnp.where(im == i, cy, accy)
        accz = jnp.where(im == i, cz, accz)
        return dist, accx, accy, accz

    _, accx, accy, accz = lax.fori_loop(
        1, M_PER, body, (dist, accx, accy, accz))
    cx_ref[...] = accx
    cy_ref[...] = accy
    cz_ref[...] = accz


def _run_fps(pt):
    out = jax.ShapeDtypeStruct((B, _MS, _ML), jnp.float32)
    return pl.pallas_call(
        _fps_body,
        out_shape=(out, out, out),
    )(pt)


_KNN_MB = 128


def _knn_body(cx_ref, cy_ref, cz_ref, pt_ref, knn_ref):
    b = pl.program_id(0)
    px = pt_ref[0, 0:1, :]
    py = pt_ref[0, 1:2, :]
    pz = pt_ref[0, 2:3, :]
    cx = cx_ref[0]
    cy = cy_ref[0]
    cz = cz_ref[0]

    dx = cx - px
    dy = cy - py
    dz = cz - pz
    d = (dx * dx + dy * dy) + dz * dz

    lane_n = lax.broadcasted_iota(jnp.int32, (_KNN_MB, N_PER), 1)
    base = b * N_PER
    inf = jnp.float32(jnp.inf)
    for k in range(NSAMPLE):
        mn = jnp.min(d, axis=1, keepdims=True)
        am = jnp.min(jnp.where(d == mn, lane_n, N_PER), axis=1,
                     keepdims=True)
        knn_ref[0, :, k:k + 1] = am + base
        d = jnp.where(lane_n == am, inf, d)


def _run_knn(cxT, cyT, czT, ptb):
    grid = (B, M_PER // _KNN_MB)
    cspec = pl.BlockSpec((1, _KNN_MB, 1), lambda b, m: (b, m, 0))
    return pl.pallas_call(
        _knn_body,
        grid=grid,
        in_specs=[
            cspec, cspec, cspec,
            pl.BlockSpec((1, 3, N_PER), lambda b, m: (b, 0, 0)),
        ],
        out_specs=pl.BlockSpec((1, _KNN_MB, NSAMPLE), lambda b, m: (b, m, 0)),
        out_shape=jax.ShapeDtypeStruct((B, M_PER, NSAMPLE), jnp.int32),
        compiler_params=pltpu.CompilerParams(
            dimension_semantics=("parallel", "parallel")),
    )(cxT, cyT, czT, ptb)


_PROJ_RB = 1024


def _proj_body(f_ref, w_ref, y_ref):
    y_ref[...] = jnp.dot(f_ref[...], w_ref[...],
                         preferred_element_type=jnp.float32)


def _run_proj(feats, wt):
    n = feats.shape[0]
    grid = (n // _PROJ_RB,)
    return pl.pallas_call(
        _proj_body,
        grid=grid,
        in_specs=[
            pl.BlockSpec((_PROJ_RB, KPAD), lambda i: (i, 0)),
            pl.BlockSpec((KPAD, OUT_C), lambda i: (0, 0)),
        ],
        out_specs=pl.BlockSpec((_PROJ_RB, OUT_C), lambda i: (i, 0)),
        out_shape=jax.ShapeDtypeStruct((n, OUT_C), jnp.float32),
        compiler_params=pltpu.CompilerParams(
            dimension_semantics=("parallel",)),
    )(feats, wt)


_SC_CHUNK = 128


def _run_sc_gather(table, idx):
    total = idx.shape[0]
    info = plsc.get_sparse_core_info()
    nw = info.num_cores * info.num_subcores
    per_w = total // nw
    nchunk = per_w // _SC_CHUNK
    mesh = plsc.VectorSubcoreMesh(core_axis_name="c", subcore_axis_name="s")

    @functools.partial(
        pl.kernel,
        mesh=mesh,
        out_type=jax.ShapeDtypeStruct((total, OUT_C), jnp.float32),
        scratch_types=[
            pltpu.VMEM((_SC_CHUNK,), jnp.int32),
            pltpu.VMEM((_SC_CHUNK, OUT_C), jnp.float32),
            pltpu.SemaphoreType.DMA,
        ],
    )
    def gather_k(table_hbm, idx_hbm, out_hbm, idx_v, rows_v, sem):
        wid = lax.axis_index("s") * info.num_cores + lax.axis_index("c")
        base = wid * per_w

        def body(j, carry):
            off = base + j * _SC_CHUNK
            pltpu.sync_copy(idx_hbm.at[pl.ds(off, _SC_CHUNK)], idx_v)
            pltpu.async_copy(table_hbm.at[idx_v], rows_v, sem).wait()
            pltpu.sync_copy(rows_v, out_hbm.at[pl.ds(off, _SC_CHUNK)])
            return carry

        lax.fori_loop(0, nchunk, body, 0)

    return gather_k(table, idx)


_LN_MB = 128


def _ln_body(g_ref, cx_ref, cy_ref, cz_ref, w3_ref, gam_ref, bet_ref, o_ref):
    cx = cx_ref[0]
    cy = cy_ref[0]
    cz = cz_ref[0]
    wx = w3_ref[0:1, :]
    wy = w3_ref[1:2, :]
    wz = w3_ref[2:3, :]
    projc = cx * wx + cy * wy + cz * wz
    gam = gam_ref[0:1, :]
    bet = bet_ref[0:1, :]

    acc = jnp.full((_LN_MB, OUT_C), -jnp.inf, dtype=jnp.float32)
    for k in range(NSAMPLE):
        z = g_ref[0, :, k] - projc
        mu = jnp.mean(z, axis=1, keepdims=True)
        zc = z - mu
        var = jnp.mean(zc * zc, axis=1, keepdims=True)
        y = zc * lax.rsqrt(var + EPS) * gam + bet
        acc = jnp.maximum(acc, y)
    o_ref[...] = jnp.maximum(acc, 0.0)


def _run_ln(g4, cxT, cyT, czT, w3, gamma2, beta2):
    grid = (B, M_PER // _LN_MB)
    cspec = pl.BlockSpec((1, _LN_MB, 1), lambda b, m: (b, m, 0))
    vspec = pl.BlockSpec((1, OUT_C), lambda b, m: (0, 0))
    return pl.pallas_call(
        _ln_body,
        grid=grid,
        in_specs=[
            pl.BlockSpec((1, _LN_MB, NSAMPLE, OUT_C), lambda b, m: (b, m, 0, 0)),
            cspec, cspec, cspec,
            pl.BlockSpec((3, OUT_C), lambda b, m: (0, 0)),
            vspec, vspec,
        ],
        out_specs=pl.BlockSpec((_LN_MB, OUT_C), lambda b, m: (b * (M_PER // _LN_MB) + m, 0)),
        out_shape=jax.ShapeDtypeStruct((B * M_PER, OUT_C), jnp.float32),
        compiler_params=pltpu.CompilerParams(
            dimension_semantics=("parallel", "parallel")),
    )(g4, cxT, cyT, czT, w3, gamma2, beta2)


def kernel(p, x, o, W, gamma, beta):
    pb = p.reshape(B, N_PER, 3)
    pt = pb.reshape(B, _FS, _FL, 3).transpose(3, 0, 1, 2)
    ptb = pb.transpose(0, 2, 1)
    cx, cy, cz = (a.reshape(B, M_PER) for a in _run_fps(pt))
    cxT, cyT, czT = cx[..., None], cy[..., None], cz[..., None]

    knn = _run_knn(cxT, cyT, czT, ptb)

    feats = jnp.concatenate([p, x], axis=1)
    feats = jnp.pad(feats, ((0, 0), (0, KPAD - feats.shape[1])))
    wt = jnp.pad(W, ((0, 0), (0, KPAD - W.shape[1]))).T
    y_all = _run_proj(feats, wt)

    idx_flat = knn.reshape(B * NSAMPLE * M_PER)
    grouped = _run_sc_gather(y_all, idx_flat)
    g4 = grouped.reshape(B, M_PER, NSAMPLE, OUT_C)

    w3 = W[:, :3].T
    x_out = _run_ln(g4, cxT, cyT, czT, w3, gamma[None, :], beta[None, :])

    n_p = jnp.stack([cx, cy, cz], axis=-1).reshape(B * M_PER, 3)
    n_o = (jnp.arange(1, B + 1) * M_PER).astype(jnp.int32)
    return (n_p, x_out, n_o)


# speedup vs baseline: 13.1384x; 1.0272x over previous
import functools

import jax
import jax.numpy as jnp
from jax import lax
from jax.experimental import pallas as pl
from jax.experimental.pallas import tpu as pltpu
from jax.experimental.pallas import tpu_sc as plsc

B = 4
N_PER = 4096
STRIDE = 4
NSAMPLE = 16
IN_C = 256
OUT_C = 512
M_PER = N_PER // STRIDE
EPS = 1e-5
KPAD = 384


_FS = 8
_FL = N_PER // _FS
_MS = 8
_ML = M_PER // _MS


def _fps_body(pt_ref, cx_ref, cy_ref, cz_ref):
    px = pt_ref[0]
    py = pt_ref[1]
    pz = pt_ref[2]

    io = (lax.broadcasted_iota(jnp.int32, (B, _FS, _FL), 1) * _FL
          + lax.broadcasted_iota(jnp.int32, (B, _FS, _FL), 2))
    im = (lax.broadcasted_iota(jnp.int32, (B, _MS, _ML), 1) * _ML
          + lax.broadcasted_iota(jnp.int32, (B, _MS, _ML), 2))

    def _rmax(a):
        return jnp.max(jnp.max(a, axis=2, keepdims=True), axis=1,
                       keepdims=True)

    def _rmin(a):
        return jnp.min(jnp.min(a, axis=2, keepdims=True), axis=1,
                       keepdims=True)

    def _rsum(a):
        return jnp.sum(jnp.sum(a, axis=2, keepdims=True), axis=1,
                       keepdims=True)

    c0x = px[:, 0:1, 0:1]
    c0y = py[:, 0:1, 0:1]
    c0z = pz[:, 0:1, 0:1]
    dx = px - c0x
    dy = py - c0y
    dz = pz - c0z
    dist = (dx * dx + dy * dy) + dz * dz

    accx = jnp.where(im == 0, c0x, 0.0)
    accy = jnp.where(im == 0, c0y, 0.0)
    accz = jnp.where(im == 0, c0z, 0.0)

    def body(i, state):
        dist, accx, accy, accz = state
        mx = _rmax(dist)
        nxt = _rmin(jnp.where(dist == mx, io, N_PER))
        onehot = io == nxt
        cx = _rsum(jnp.where(onehot, px, 0.0))
        cy = _rsum(jnp.where(onehot, py, 0.0))
        cz = _rsum(jnp.where(onehot, pz, 0.0))
        dx = px - cx
        dy = py - cy
        dz = pz - cz
        d = (dx * dx + dy * dy) + dz * dz
        dist = jnp.minimum(dist, d)
        accx = jnp.where(im == i, cx, accx)
        accy = jnp.where(im == i, cy, accy)
        accz = jnp.where(im == i, cz, accz)
        return dist, accx, accy, accz

    def body2(j, state):
        state = body(2 + 2 * j, state)
        return body(3 + 2 * j, state)

    state = body(1, (dist, accx, accy, accz))
    _, accx, accy, accz = lax.fori_loop(0, (M_PER - 2) // 2, body2, state)
    cx_ref[...] = accx
    cy_ref[...] = accy
    cz_ref[...] = accz


def _run_fps(pt):
    out = jax.ShapeDtypeStruct((B, _MS, _ML), jnp.float32)
    return pl.pallas_call(
        _fps_body,
        out_shape=(out, out, out),
    )(pt)


_KNN_MB = 128


def _knn_body(cx_ref, cy_ref, cz_ref, pt_ref, knn_ref):
    b = pl.program_id(0)
    px = pt_ref[0, 0:1, :]
    py = pt_ref[0, 1:2, :]
    pz = pt_ref[0, 2:3, :]
    cx = cx_ref[0]
    cy = cy_ref[0]
    cz = cz_ref[0]

    dx = cx - px
    dy = cy - py
    dz = cz - pz
    d = (dx * dx + dy * dy) + dz * dz

    lane_n = lax.broadcasted_iota(jnp.int32, (_KNN_MB, N_PER), 1)
    base = b * N_PER
    inf = jnp.float32(jnp.inf)
    for k in range(NSAMPLE):
        mn = jnp.min(d, axis=1, keepdims=True)
        am = jnp.min(jnp.where(d == mn, lane_n, N_PER), axis=1,
                     keepdims=True)
        knn_ref[0, :, k:k + 1] = am + base
        d = jnp.where(lane_n == am, inf, d)


def _run_knn(cxT, cyT, czT, ptb):
    grid = (B, M_PER // _KNN_MB)
    cspec = pl.BlockSpec((1, _KNN_MB, 1), lambda b, m: (b, m, 0))
    return pl.pallas_call(
        _knn_body,
        grid=grid,
        in_specs=[
            cspec, cspec, cspec,
            pl.BlockSpec((1, 3, N_PER), lambda b, m: (b, 0, 0)),
        ],
        out_specs=pl.BlockSpec((1, _KNN_MB, NSAMPLE), lambda b, m: (b, m, 0)),
        out_shape=jax.ShapeDtypeStruct((B, M_PER, NSAMPLE), jnp.int32),
        compiler_params=pltpu.CompilerParams(
            dimension_semantics=("parallel", "parallel")),
    )(cxT, cyT, czT, ptb)


_PROJ_RB = 1024


def _proj_body(x_ref, p_ref, wxt_ref, w3_ref, y_ref):
    y = jnp.dot(x_ref[...], wxt_ref[...], preferred_element_type=jnp.float32)
    y += p_ref[:, 0:1] * w3_ref[0:1, :]
    y += p_ref[:, 1:2] * w3_ref[1:2, :]
    y += p_ref[:, 2:3] * w3_ref[2:3, :]
    y_ref[...] = y


def _run_proj(x, p, wxt, w3):
    n = x.shape[0]
    grid = (n // _PROJ_RB,)
    return pl.pallas_call(
        _proj_body,
        grid=grid,
        in_specs=[
            pl.BlockSpec((_PROJ_RB, IN_C), lambda i: (i, 0)),
            pl.BlockSpec((_PROJ_RB, 3), lambda i: (i, 0)),
            pl.BlockSpec((IN_C, OUT_C), lambda i: (0, 0)),
            pl.BlockSpec((3, OUT_C), lambda i: (0, 0)),
        ],
        out_specs=pl.BlockSpec((_PROJ_RB, OUT_C), lambda i: (i, 0)),
        out_shape=jax.ShapeDtypeStruct((n, OUT_C), jnp.float32),
        compiler_params=pltpu.CompilerParams(
            dimension_semantics=("parallel",)),
    )(x, p, wxt, w3)


_SC_CHUNK = 128


def _run_sc_gather(table, idx):
    total = idx.shape[0]
    info = plsc.get_sparse_core_info()
    nw = info.num_cores * info.num_subcores
    per_w = total // nw
    nchunk = per_w // _SC_CHUNK
    mesh = plsc.VectorSubcoreMesh(core_axis_name="c", subcore_axis_name="s")

    @functools.partial(
        pl.kernel,
        mesh=mesh,
        out_type=jax.ShapeDtypeStruct((total, OUT_C), jnp.float32),
        scratch_types=[
            pltpu.VMEM((_SC_CHUNK,), jnp.int32),
            pltpu.VMEM((_SC_CHUNK, OUT_C), jnp.float32),
            pltpu.SemaphoreType.DMA,
        ],
    )
    def gather_k(table_hbm, idx_hbm, out_hbm, idx_v, rows_v, sem):
        wid = lax.axis_index("s") * info.num_cores + lax.axis_index("c")
        base = wid * per_w

        def body(j, carry):
            off = base + j * _SC_CHUNK
            pltpu.sync_copy(idx_hbm.at[pl.ds(off, _SC_CHUNK)], idx_v)
            pltpu.async_copy(table_hbm.at[idx_v], rows_v, sem).wait()
            pltpu.sync_copy(rows_v, out_hbm.at[pl.ds(off, _SC_CHUNK)])
            return carry

        lax.fori_loop(0, nchunk, body, 0)

    return gather_k(table, idx)


_LN_MB = 128


def _ln_body(g_ref, cx_ref, cy_ref, cz_ref, w3_ref, gam_ref, bet_ref, o_ref):
    cx = cx_ref[0]
    cy = cy_ref[0]
    cz = cz_ref[0]
    wx = w3_ref[0:1, :]
    wy = w3_ref[1:2, :]
    wz = w3_ref[2:3, :]
    projc = cx * wx + cy * wy + cz * wz
    gam = gam_ref[0:1, :]
    bet = bet_ref[0:1, :]

    acc = jnp.full((_LN_MB, OUT_C), -jnp.inf, dtype=jnp.float32)
    for k in range(NSAMPLE):
        z = g_ref[0, :, k] - projc
        mu = jnp.mean(z, axis=1, keepdims=True)
        zc = z - mu
        var = jnp.mean(zc * zc, axis=1, keepdims=True)
        y = zc * lax.rsqrt(var + EPS) * gam + bet
        acc = jnp.maximum(acc, y)
    o_ref[...] = jnp.maximum(acc, 0.0)


def _run_ln(g4, cxT, cyT, czT, w3, gamma2, beta2):
    grid = (B, M_PER // _LN_MB)
    cspec = pl.BlockSpec((1, _LN_MB, 1), lambda b, m: (b, m, 0))
    vspec = pl.BlockSpec((1, OUT_C), lambda b, m: (0, 0))
    return pl.pallas_call(
        _ln_body,
        grid=grid,
        in_specs=[
            pl.BlockSpec((1, _LN_MB, NSAMPLE, OUT_C), lambda b, m: (b, m, 0, 0)),
            cspec, cspec, cspec,
            pl.BlockSpec((3, OUT_C), lambda b, m: (0, 0)),
            vspec, vspec,
        ],
        out_specs=pl.BlockSpec((_LN_MB, OUT_C), lambda b, m: (b * (M_PER // _LN_MB) + m, 0)),
        out_shape=jax.ShapeDtypeStruct((B * M_PER, OUT_C), jnp.float32),
        compiler_params=pltpu.CompilerParams(
            dimension_semantics=("parallel", "parallel")),
    )(g4, cxT, cyT, czT, w3, gamma2, beta2)


def kernel(p, x, o, W, gamma, beta):
    pb = p.reshape(B, N_PER, 3)
    pt = pb.reshape(B, _FS, _FL, 3).transpose(3, 0, 1, 2)
    ptb = pb.transpose(0, 2, 1)
    cx, cy, cz = (a.reshape(B, M_PER) for a in _run_fps(pt))
    cxT, cyT, czT = cx[..., None], cy[..., None], cz[..., None]

    knn = _run_knn(cxT, cyT, czT, ptb)

    w3 = W[:, :3].T
    wxt = W[:, 3:].T
    y_all = _run_proj(x, p, wxt, w3)

    idx_flat = knn.reshape(B * NSAMPLE * M_PER)
    grouped = _run_sc_gather(y_all, idx_flat)
    g4 = grouped.reshape(B, M_PER, NSAMPLE, OUT_C)

    x_out = _run_ln(g4, cxT, cyT, czT, w3, gamma[None, :], beta[None, :])

    n_p = jnp.stack([cx, cy, cz], axis=-1).reshape(B * M_PER, 3)
    n_o = (jnp.arange(1, B + 1) * M_PER).astype(jnp.int32)
    return (n_p, x_out, n_o)


# speedup vs baseline: 13.8606x; 1.0550x over previous
import functools

import jax
import jax.numpy as jnp
from jax import lax
from jax.experimental import pallas as pl
from jax.experimental.pallas import tpu as pltpu
from jax.experimental.pallas import tpu_sc as plsc

B = 4
N_PER = 4096
STRIDE = 4
NSAMPLE = 16
IN_C = 256
OUT_C = 512
M_PER = N_PER // STRIDE
EPS = 1e-5
KPAD = 384


_FS = 8
_FL = N_PER // _FS
_MS = 8
_ML = M_PER // _MS


def _fps_body(pt_ref, cx_ref, cy_ref, cz_ref):
    px = pt_ref[0]
    py = pt_ref[1]
    pz = pt_ref[2]

    io = (lax.broadcasted_iota(jnp.int32, (B, _FS, _FL), 1) * _FL
          + lax.broadcasted_iota(jnp.int32, (B, _FS, _FL), 2))
    im = (lax.broadcasted_iota(jnp.int32, (B, _MS, _ML), 1) * _ML
          + lax.broadcasted_iota(jnp.int32, (B, _MS, _ML), 2))

    def _rmax(a):
        return jnp.max(jnp.max(a, axis=2, keepdims=True), axis=1,
                       keepdims=True)

    def _rmin(a):
        return jnp.min(jnp.min(a, axis=2, keepdims=True), axis=1,
                       keepdims=True)

    def _rsum(a):
        return jnp.sum(jnp.sum(a, axis=2, keepdims=True), axis=1,
                       keepdims=True)

    c0x = px[:, 0:1, 0:1]
    c0y = py[:, 0:1, 0:1]
    c0z = pz[:, 0:1, 0:1]
    dx = px - c0x
    dy = py - c0y
    dz = pz - c0z
    dist = (dx * dx + dy * dy) + dz * dz

    accx = jnp.where(im == 0, c0x, 0.0)
    accy = jnp.where(im == 0, c0y, 0.0)
    accz = jnp.where(im == 0, c0z, 0.0)

    def body(i, state):
        dist, accx, accy, accz = state
        mx = _rmax(dist)
        nxt = _rmin(jnp.where(dist == mx, io, N_PER))
        onehot = io == nxt
        cx = _rsum(jnp.where(onehot, px, 0.0))
        cy = _rsum(jnp.where(onehot, py, 0.0))
        cz = _rsum(jnp.where(onehot, pz, 0.0))
        dx = px - cx
        dy = py - cy
        dz = pz - cz
        d = (dx * dx + dy * dy) + dz * dz
        dist = jnp.minimum(dist, d)
        accx = jnp.where(im == i, cx, accx)
        accy = jnp.where(im == i, cy, accy)
        accz = jnp.where(im == i, cz, accz)
        return dist, accx, accy, accz

    def body2(j, state):
        state = body(2 + 2 * j, state)
        return body(3 + 2 * j, state)

    state = body(1, (dist, accx, accy, accz))
    _, accx, accy, accz = lax.fori_loop(0, (M_PER - 2) // 2, body2, state)
    cx_ref[...] = accx
    cy_ref[...] = accy
    cz_ref[...] = accz


def _run_fps(pt):
    out = jax.ShapeDtypeStruct((B, _MS, _ML), jnp.float32)
    return pl.pallas_call(
        _fps_body,
        out_shape=(out, out, out),
    )(pt)


_KNN_MB = 128


def _knn_body(cx_ref, cy_ref, cz_ref, pt_ref, knn_ref):
    b = pl.program_id(0)
    px = pt_ref[0, 0:1, :]
    py = pt_ref[0, 1:2, :]
    pz = pt_ref[0, 2:3, :]
    cx = cx_ref[0]
    cy = cy_ref[0]
    cz = cz_ref[0]

    dx = cx - px
    dy = cy - py
    dz = cz - pz
    d = (dx * dx + dy * dy) + dz * dz

    lane_n = lax.broadcasted_iota(jnp.int32, (_KNN_MB, N_PER), 1)
    base = b * N_PER
    inf = jnp.float32(jnp.inf)
    for k in range(NSAMPLE):
        am = jnp.argmin(d, axis=1).astype(jnp.int32)[:, None]
        knn_ref[0, :, k:k + 1] = am + base
        d = jnp.where(lane_n == am, inf, d)


def _run_knn(cxT, cyT, czT, ptb):
    grid = (B, M_PER // _KNN_MB)
    cspec = pl.BlockSpec((1, _KNN_MB, 1), lambda b, m: (b, m, 0))
    return pl.pallas_call(
        _knn_body,
        grid=grid,
        in_specs=[
            cspec, cspec, cspec,
            pl.BlockSpec((1, 3, N_PER), lambda b, m: (b, 0, 0)),
        ],
        out_specs=pl.BlockSpec((1, _KNN_MB, NSAMPLE), lambda b, m: (b, m, 0)),
        out_shape=jax.ShapeDtypeStruct((B, M_PER, NSAMPLE), jnp.int32),
        compiler_params=pltpu.CompilerParams(
            dimension_semantics=("parallel", "parallel")),
    )(cxT, cyT, czT, ptb)


_PROJ_RB = 1024


def _proj_body(x_ref, p_ref, wxt_ref, w3_ref, y_ref):
    y = jnp.dot(x_ref[...], wxt_ref[...], preferred_element_type=jnp.float32)
    y += p_ref[:, 0:1] * w3_ref[0:1, :]
    y += p_ref[:, 1:2] * w3_ref[1:2, :]
    y += p_ref[:, 2:3] * w3_ref[2:3, :]
    y_ref[...] = y


def _run_proj(x, p, wxt, w3):
    n = x.shape[0]
    grid = (n // _PROJ_RB,)
    return pl.pallas_call(
        _proj_body,
        grid=grid,
        in_specs=[
            pl.BlockSpec((_PROJ_RB, IN_C), lambda i: (i, 0)),
            pl.BlockSpec((_PROJ_RB, 3), lambda i: (i, 0)),
            pl.BlockSpec((IN_C, OUT_C), lambda i: (0, 0)),
            pl.BlockSpec((3, OUT_C), lambda i: (0, 0)),
        ],
        out_specs=pl.BlockSpec((_PROJ_RB, OUT_C), lambda i: (i, 0)),
        out_shape=jax.ShapeDtypeStruct((n, OUT_C), jnp.float32),
        compiler_params=pltpu.CompilerParams(
            dimension_semantics=("parallel",)),
    )(x, p, wxt, w3)


_SC_CHUNK = 128


def _run_sc_gather(table, idx):
    total = idx.shape[0]
    info = plsc.get_sparse_core_info()
    nw = info.num_cores * info.num_subcores
    per_w = total // nw
    nchunk = per_w // _SC_CHUNK
    mesh = plsc.VectorSubcoreMesh(core_axis_name="c", subcore_axis_name="s")

    @functools.partial(
        pl.kernel,
        mesh=mesh,
        out_type=jax.ShapeDtypeStruct((total, OUT_C), jnp.float32),
        scratch_types=[
            pltpu.VMEM((_SC_CHUNK,), jnp.int32),
            pltpu.VMEM((_SC_CHUNK, OUT_C), jnp.float32),
            pltpu.SemaphoreType.DMA,
        ],
    )
    def gather_k(table_hbm, idx_hbm, out_hbm, idx_v, rows_v, sem):
        wid = lax.axis_index("s") * info.num_cores + lax.axis_index("c")
        base = wid * per_w

        def body(j, carry):
            off = base + j * _SC_CHUNK
            pltpu.sync_copy(idx_hbm.at[pl.ds(off, _SC_CHUNK)], idx_v)
            pltpu.async_copy(table_hbm.at[idx_v], rows_v, sem).wait()
            pltpu.sync_copy(rows_v, out_hbm.at[pl.ds(off, _SC_CHUNK)])
            return carry

        lax.fori_loop(0, nchunk, body, 0)

    return gather_k(table, idx)


_LN_MB = 128


def _ln_body(g_ref, cx_ref, cy_ref, cz_ref, w3_ref, gam_ref, bet_ref, o_ref):
    cx = cx_ref[0]
    cy = cy_ref[0]
    cz = cz_ref[0]
    wx = w3_ref[0:1, :]
    wy = w3_ref[1:2, :]
    wz = w3_ref[2:3, :]
    projc = cx * wx + cy * wy + cz * wz
    gam = gam_ref[0:1, :]
    bet = bet_ref[0:1, :]

    acc = jnp.full((_LN_MB, OUT_C), -jnp.inf, dtype=jnp.float32)
    for k in range(NSAMPLE):
        z = g_ref[0, :, k] - projc
        mu = jnp.mean(z, axis=1, keepdims=True)
        zc = z - mu
        var = jnp.mean(zc * zc, axis=1, keepdims=True)
        y = zc * lax.rsqrt(var + EPS) * gam + bet
        acc = jnp.maximum(acc, y)
    o_ref[...] = jnp.maximum(acc, 0.0)


def _run_ln(g4, cxT, cyT, czT, w3, gamma2, beta2):
    grid = (B, M_PER // _LN_MB)
    cspec = pl.BlockSpec((1, _LN_MB, 1), lambda b, m: (b, m, 0))
    vspec = pl.BlockSpec((1, OUT_C), lambda b, m: (0, 0))
    return pl.pallas_call(
        _ln_body,
        grid=grid,
        in_specs=[
            pl.BlockSpec((1, _LN_MB, NSAMPLE, OUT_C), lambda b, m: (b, m, 0, 0)),
            cspec, cspec, cspec,
            pl.BlockSpec((3, OUT_C), lambda b, m: (0, 0)),
            vspec, vspec,
        ],
        out_specs=pl.BlockSpec((_LN_MB, OUT_C), lambda b, m: (b * (M_PER // _LN_MB) + m, 0)),
        out_shape=jax.ShapeDtypeStruct((B * M_PER, OUT_C), jnp.float32),
        compiler_params=pltpu.CompilerParams(
            dimension_semantics=("parallel", "parallel")),
    )(g4, cxT, cyT, czT, w3, gamma2, beta2)


def kernel(p, x, o, W, gamma, beta):
    pb = p.reshape(B, N_PER, 3)
    pt = pb.reshape(B, _FS, _FL, 3).transpose(3, 0, 1, 2)
    ptb = pb.transpose(0, 2, 1)
    cx, cy, cz = (a.reshape(B, M_PER) for a in _run_fps(pt))
    cxT, cyT, czT = cx[..., None], cy[..., None], cz[..., None]

    knn = _run_knn(cxT, cyT, czT, ptb)

    w3 = W[:, :3].T
    wxt = W[:, 3:].T
    y_all = _run_proj(x, p, wxt, w3)

    idx_flat = knn.reshape(B * NSAMPLE * M_PER)
    grouped = _run_sc_gather(y_all, idx_flat)
    g4 = grouped.reshape(B, M_PER, NSAMPLE, OUT_C)

    x_out = _run_ln(g4, cxT, cyT, czT, w3, gamma[None, :], beta[None, :])

    n_p = jnp.stack([cx, cy, cz], axis=-1).reshape(B * M_PER, 3)
    n_o = (jnp.arange(1, B + 1) * M_PER).astype(jnp.int32)
    return (n_p, x_out, n_o)


# speedup vs baseline: 14.3629x; 1.0362x over previous
import functools

import jax
import jax.numpy as jnp
from jax import lax
from jax.experimental import pallas as pl
from jax.experimental.pallas import tpu as pltpu
from jax.experimental.pallas import tpu_sc as plsc

B = 4
N_PER = 4096
STRIDE = 4
NSAMPLE = 16
IN_C = 256
OUT_C = 512
M_PER = N_PER // STRIDE
EPS = 1e-5
KPAD = 384


_FS = 8
_FL = N_PER // _FS
_MS = 8
_ML = M_PER // _MS


def _fps_body(pt_ref, cx_ref, cy_ref, cz_ref):
    px = pt_ref[0]
    py = pt_ref[1]
    pz = pt_ref[2]

    io = (lax.broadcasted_iota(jnp.int32, (B, _FS, _FL), 1) * _FL
          + lax.broadcasted_iota(jnp.int32, (B, _FS, _FL), 2))
    im = (lax.broadcasted_iota(jnp.int32, (B, _MS, _ML), 1) * _ML
          + lax.broadcasted_iota(jnp.int32, (B, _MS, _ML), 2))

    def _rmax(a):
        return jnp.max(jnp.max(a, axis=2, keepdims=True), axis=1,
                       keepdims=True)

    def _rmin(a):
        return jnp.min(jnp.min(a, axis=2, keepdims=True), axis=1,
                       keepdims=True)

    def _rsum(a):
        return jnp.sum(jnp.sum(a, axis=2, keepdims=True), axis=1,
                       keepdims=True)

    c0x = px[:, 0:1, 0:1]
    c0y = py[:, 0:1, 0:1]
    c0z = pz[:, 0:1, 0:1]
    dx = px - c0x
    dy = py - c0y
    dz = pz - c0z
    dist = (dx * dx + dy * dy) + dz * dz

    accx = jnp.where(im == 0, c0x, 0.0)
    accy = jnp.where(im == 0, c0y, 0.0)
    accz = jnp.where(im == 0, c0z, 0.0)

    def body(i, state):
        dist, accx, accy, accz = state
        mx = _rmax(dist)
        nxt = _rmin(jnp.where(dist == mx, io, N_PER))
        onehot = io == nxt
        cx = _rsum(jnp.where(onehot, px, 0.0))
        cy = _rsum(jnp.where(onehot, py, 0.0))
        cz = _rsum(jnp.where(onehot, pz, 0.0))
        dx = px - cx
        dy = py - cy
        dz = pz - cz
        d = (dx * dx + dy * dy) + dz * dz
        dist = jnp.minimum(dist, d)
        accx = jnp.where(im == i, cx, accx)
        accy = jnp.where(im == i, cy, accy)
        accz = jnp.where(im == i, cz, accz)
        return dist, accx, accy, accz

    def body2(j, state):
        state = body(2 + 2 * j, state)
        return body(3 + 2 * j, state)

    state = body(1, (dist, accx, accy, accz))
    _, accx, accy, accz = lax.fori_loop(0, (M_PER - 2) // 2, body2, state)
    cx_ref[...] = accx
    cy_ref[...] = accy
    cz_ref[...] = accz


def _run_fps(pt):
    out = jax.ShapeDtypeStruct((B, _MS, _ML), jnp.float32)
    return pl.pallas_call(
        _fps_body,
        out_shape=(out, out, out),
    )(pt)


_KNN_MB = 128


def _knn_body(cx_ref, cy_ref, cz_ref, pt_ref, knn_ref):
    b = pl.program_id(0)
    px = pt_ref[0, 0:1, :]
    py = pt_ref[0, 1:2, :]
    pz = pt_ref[0, 2:3, :]
    cx = cx_ref[0]
    cy = cy_ref[0]
    cz = cz_ref[0]

    dx = cx - px
    dy = cy - py
    dz = cz - pz
    d = (dx * dx + dy * dy) + dz * dz

    lane_n = lax.broadcasted_iota(jnp.int32, (_KNN_MB, N_PER), 1)
    base = b * N_PER
    inf = jnp.float32(jnp.inf)
    for k in range(NSAMPLE):
        am = jnp.argmin(d, axis=1).astype(jnp.int32)[:, None]
        knn_ref[0, :, k:k + 1] = am + base
        d = jnp.where(lane_n == am, inf, d)


def _run_knn(cxT, cyT, czT, ptb):
    grid = (B, M_PER // _KNN_MB)
    cspec = pl.BlockSpec((1, _KNN_MB, 1), lambda b, m: (b, m, 0))
    return pl.pallas_call(
        _knn_body,
        grid=grid,
        in_specs=[
            cspec, cspec, cspec,
            pl.BlockSpec((1, 3, N_PER), lambda b, m: (b, 0, 0)),
        ],
        out_specs=pl.BlockSpec((1, _KNN_MB, NSAMPLE), lambda b, m: (b, m, 0)),
        out_shape=jax.ShapeDtypeStruct((B, M_PER, NSAMPLE), jnp.int32),
        compiler_params=pltpu.CompilerParams(
            dimension_semantics=("parallel", "parallel")),
    )(cxT, cyT, czT, ptb)


_PROJ_RB = 1024


def _proj_body(x_ref, p_ref, wxt_ref, w3_ref, y_ref):
    y = jnp.dot(x_ref[...], wxt_ref[...], preferred_element_type=jnp.float32)
    y += p_ref[:, 0:1] * w3_ref[0:1, :]
    y += p_ref[:, 1:2] * w3_ref[1:2, :]
    y += p_ref[:, 2:3] * w3_ref[2:3, :]
    y_ref[...] = y


def _run_proj(x, p, wxt, w3):
    n = x.shape[0]
    grid = (n // _PROJ_RB,)
    return pl.pallas_call(
        _proj_body,
        grid=grid,
        in_specs=[
            pl.BlockSpec((_PROJ_RB, IN_C), lambda i: (i, 0)),
            pl.BlockSpec((_PROJ_RB, 3), lambda i: (i, 0)),
            pl.BlockSpec((IN_C, OUT_C), lambda i: (0, 0)),
            pl.BlockSpec((3, OUT_C), lambda i: (0, 0)),
        ],
        out_specs=pl.BlockSpec((_PROJ_RB, OUT_C), lambda i: (i, 0)),
        out_shape=jax.ShapeDtypeStruct((n, OUT_C), jnp.float32),
        compiler_params=pltpu.CompilerParams(
            dimension_semantics=("parallel",)),
    )(x, p, wxt, w3)


_SC_CHUNK = 128


def _run_sc_gather(table, idx):
    total = idx.shape[0]
    info = plsc.get_sparse_core_info()
    nw = info.num_cores * info.num_subcores
    per_w = total // nw
    nchunk = per_w // _SC_CHUNK
    mesh = plsc.VectorSubcoreMesh(core_axis_name="c", subcore_axis_name="s")

    @functools.partial(
        pl.kernel,
        mesh=mesh,
        out_type=jax.ShapeDtypeStruct((total, OUT_C), jnp.float32),
        scratch_types=[
            pltpu.VMEM((_SC_CHUNK,), jnp.int32),
            pltpu.VMEM((_SC_CHUNK, OUT_C), jnp.float32),
            pltpu.SemaphoreType.DMA,
        ],
    )
    def gather_k(table_hbm, idx_hbm, out_hbm, idx_v, rows_v, sem):
        wid = lax.axis_index("s") * info.num_cores + lax.axis_index("c")
        base = wid * per_w

        def body(j, carry):
            off = base + j * _SC_CHUNK
            pltpu.sync_copy(idx_hbm.at[pl.ds(off, _SC_CHUNK)], idx_v)
            pltpu.async_copy(table_hbm.at[idx_v], rows_v, sem).wait()
            pltpu.sync_copy(rows_v, out_hbm.at[pl.ds(off, _SC_CHUNK)])
            return carry

        lax.fori_loop(0, nchunk, body, 0)

    return gather_k(table, idx)


_LN_MB = 128


def _ln_body(g_ref, cx_ref, cy_ref, cz_ref, w3_ref, gam_ref, bet_ref, o_ref):
    cx = cx_ref[0]
    cy = cy_ref[0]
    cz = cz_ref[0]
    wx = w3_ref[0:1, :]
    wy = w3_ref[1:2, :]
    wz = w3_ref[2:3, :]
    projc = cx * wx + cy * wy + cz * wz
    gam = gam_ref[0:1, :]
    bet = bet_ref[0:1, :]

    acc = jnp.full((_LN_MB, OUT_C), -jnp.inf, dtype=jnp.float32)
    for k in range(NSAMPLE):
        z = g_ref[0, :, k] - projc
        mu = jnp.mean(z, axis=1, keepdims=True)
        zc = z - mu
        var = jnp.mean(zc * zc, axis=1, keepdims=True)
        y = zc * lax.rsqrt(var + EPS) * gam + bet
        acc = jnp.maximum(acc, y)
    o_ref[...] = jnp.maximum(acc, 0.0)


def _run_ln(g4, cxT, cyT, czT, w3, gamma2, beta2):
    nb = g4.shape[0]
    grid = (nb, M_PER // _LN_MB)
    cspec = pl.BlockSpec((1, _LN_MB, 1), lambda b, m: (b, m, 0))
    vspec = pl.BlockSpec((1, OUT_C), lambda b, m: (0, 0))
    return pl.pallas_call(
        _ln_body,
        grid=grid,
        in_specs=[
            pl.BlockSpec((1, _LN_MB, NSAMPLE, OUT_C), lambda b, m: (b, m, 0, 0)),
            cspec, cspec, cspec,
            pl.BlockSpec((3, OUT_C), lambda b, m: (0, 0)),
            vspec, vspec,
        ],
        out_specs=pl.BlockSpec((_LN_MB, OUT_C), lambda b, m: (b * (M_PER // _LN_MB) + m, 0)),
        out_shape=jax.ShapeDtypeStruct((nb * M_PER, OUT_C), jnp.float32),
        compiler_params=pltpu.CompilerParams(
            dimension_semantics=("parallel", "parallel")),
    )(g4, cxT, cyT, czT, w3, gamma2, beta2)


def kernel(p, x, o, W, gamma, beta):
    pb = p.reshape(B, N_PER, 3)
    pt = pb.reshape(B, _FS, _FL, 3).transpose(3, 0, 1, 2)
    ptb = pb.transpose(0, 2, 1)
    cx, cy, cz = (a.reshape(B, M_PER) for a in _run_fps(pt))
    cxT, cyT, czT = cx[..., None], cy[..., None], cz[..., None]

    knn = _run_knn(cxT, cyT, czT, ptb)

    w3 = W[:, :3].T
    wxt = W[:, 3:].T
    y_all = _run_proj(x, p, wxt, w3)

    idx_flat = knn.reshape(B * NSAMPLE * M_PER)
    half = B * NSAMPLE * M_PER // 2
    hb = B // 2
    gam2, bet2 = gamma[None, :], beta[None, :]
    grouped0 = _run_sc_gather(y_all, idx_flat[:half])
    grouped1 = _run_sc_gather(y_all, idx_flat[half:])
    g40 = grouped0.reshape(hb, M_PER, NSAMPLE, OUT_C)
    g41 = grouped1.reshape(hb, M_PER, NSAMPLE, OUT_C)
    x_out0 = _run_ln(g40, cxT[:hb], cyT[:hb], czT[:hb], w3, gam2, bet2)
    x_out1 = _run_ln(g41, cxT[hb:], cyT[hb:], czT[hb:], w3, gam2, bet2)
    x_out = jnp.concatenate([x_out0, x_out1], axis=0)

    n_p = jnp.stack([cx, cy, cz], axis=-1).reshape(B * M_PER, 3)
    n_o = (jnp.arange(1, B + 1) * M_PER).astype(jnp.int32)
    return (n_p, x_out, n_o)


# speedup vs baseline: 14.6614x; 1.0208x over previous
import functools

import jax
import jax.numpy as jnp
from jax import lax
from jax.experimental import pallas as pl
from jax.experimental.pallas import tpu as pltpu
from jax.experimental.pallas import tpu_sc as plsc

B = 4
N_PER = 4096
STRIDE = 4
NSAMPLE = 16
IN_C = 256
OUT_C = 512
M_PER = N_PER // STRIDE
EPS = 1e-5
KPAD = 384


_FS = 8
_FL = N_PER // _FS
_MS = 8
_ML = M_PER // _MS


def _fps_body(pt_ref, cx_ref, cy_ref, cz_ref):
    px = pt_ref[0]
    py = pt_ref[1]
    pz = pt_ref[2]

    io = (lax.broadcasted_iota(jnp.int32, (B, _FS, _FL), 1) * _FL
          + lax.broadcasted_iota(jnp.int32, (B, _FS, _FL), 2))
    im = (lax.broadcasted_iota(jnp.int32, (B, _MS, _ML), 1) * _ML
          + lax.broadcasted_iota(jnp.int32, (B, _MS, _ML), 2))

    def _rmax(a):
        return jnp.max(jnp.max(a, axis=2, keepdims=True), axis=1,
                       keepdims=True)

    def _rmin(a):
        return jnp.min(jnp.min(a, axis=2, keepdims=True), axis=1,
                       keepdims=True)

    def _rsum(a):
        return jnp.sum(jnp.sum(a, axis=2, keepdims=True), axis=1,
                       keepdims=True)

    c0x = px[:, 0:1, 0:1]
    c0y = py[:, 0:1, 0:1]
    c0z = pz[:, 0:1, 0:1]
    dx = px - c0x
    dy = py - c0y
    dz = pz - c0z
    dist = (dx * dx + dy * dy) + dz * dz

    accx = jnp.where(im == 0, c0x, 0.0)
    accy = jnp.where(im == 0, c0y, 0.0)
    accz = jnp.where(im == 0, c0z, 0.0)

    def body(i, state):
        dist, accx, accy, accz = state
        mx = _rmax(dist)
        nxt = _rmin(jnp.where(dist == mx, io, N_PER))
        onehot = io == nxt
        cx = _rsum(jnp.where(onehot, px, 0.0))
        cy = _rsum(jnp.where(onehot, py, 0.0))
        cz = _rsum(jnp.where(onehot, pz, 0.0))
        dx = px - cx
        dy = py - cy
        dz = pz - cz
        d = (dx * dx + dy * dy) + dz * dz
        dist = jnp.minimum(dist, d)
        accx = jnp.where(im == i, cx, accx)
        accy = jnp.where(im == i, cy, accy)
        accz = jnp.where(im == i, cz, accz)
        return dist, accx, accy, accz

    def body2(j, state):
        state = body(2 + 2 * j, state)
        return body(3 + 2 * j, state)

    state = body(1, (dist, accx, accy, accz))
    _, accx, accy, accz = lax.fori_loop(0, (M_PER - 2) // 2, body2, state)
    cx_ref[...] = accx
    cy_ref[...] = accy
    cz_ref[...] = accz


def _run_fps(pt):
    out = jax.ShapeDtypeStruct((B, _MS, _ML), jnp.float32)
    return pl.pallas_call(
        _fps_body,
        out_shape=(out, out, out),
    )(pt)


_KNN_MB = 128


def _knn_body(cx_ref, cy_ref, cz_ref, pt_ref, knn_ref):
    b = pl.program_id(0)
    px = pt_ref[0, 0:1, :]
    py = pt_ref[0, 1:2, :]
    pz = pt_ref[0, 2:3, :]
    cx = cx_ref[0]
    cy = cy_ref[0]
    cz = cz_ref[0]

    dx = cx - px
    dy = cy - py
    dz = cz - pz
    d = (dx * dx + dy * dy) + dz * dz

    lane_n = lax.broadcasted_iota(jnp.int32, (_KNN_MB, N_PER), 1)
    base = b * N_PER
    inf = jnp.float32(jnp.inf)
    for k in range(NSAMPLE):
        am = jnp.argmin(d, axis=1).astype(jnp.int32)[:, None]
        knn_ref[0, :, k:k + 1] = am + base
        d = jnp.where(lane_n == am, inf, d)


def _run_knn(cxT, cyT, czT, ptb):
    grid = (B, M_PER // _KNN_MB)
    cspec = pl.BlockSpec((1, _KNN_MB, 1), lambda b, m: (b, m, 0))
    return pl.pallas_call(
        _knn_body,
        grid=grid,
        in_specs=[
            cspec, cspec, cspec,
            pl.BlockSpec((1, 3, N_PER), lambda b, m: (b, 0, 0)),
        ],
        out_specs=pl.BlockSpec((1, _KNN_MB, NSAMPLE), lambda b, m: (b, m, 0)),
        out_shape=jax.ShapeDtypeStruct((B, M_PER, NSAMPLE), jnp.int32),
        compiler_params=pltpu.CompilerParams(
            dimension_semantics=("parallel", "parallel")),
    )(cxT, cyT, czT, ptb)


_PROJ_RB = 1024


def _proj_body(x_ref, p_ref, wxt_ref, w3_ref, y_ref):
    y = jnp.dot(x_ref[...], wxt_ref[...], preferred_element_type=jnp.float32)
    y += p_ref[:, 0:1] * w3_ref[0:1, :]
    y += p_ref[:, 1:2] * w3_ref[1:2, :]
    y += p_ref[:, 2:3] * w3_ref[2:3, :]
    y_ref[...] = y


def _run_proj(x, p, wxt, w3):
    n = x.shape[0]
    grid = (n // _PROJ_RB,)
    return pl.pallas_call(
        _proj_body,
        grid=grid,
        in_specs=[
            pl.BlockSpec((_PROJ_RB, IN_C), lambda i: (i, 0)),
            pl.BlockSpec((_PROJ_RB, 3), lambda i: (i, 0)),
            pl.BlockSpec((IN_C, OUT_C), lambda i: (0, 0)),
            pl.BlockSpec((3, OUT_C), lambda i: (0, 0)),
        ],
        out_specs=pl.BlockSpec((_PROJ_RB, OUT_C), lambda i: (i, 0)),
        out_shape=jax.ShapeDtypeStruct((n, OUT_C), jnp.float32),
        compiler_params=pltpu.CompilerParams(
            dimension_semantics=("parallel",)),
    )(x, p, wxt, w3)


_SC_CHUNK = 128


def _run_sc_gather(table, idx):
    total = idx.shape[0]
    info = plsc.get_sparse_core_info()
    nw = info.num_cores * info.num_subcores
    per_w = total // nw
    nchunk = per_w // _SC_CHUNK
    mesh = plsc.VectorSubcoreMesh(core_axis_name="c", subcore_axis_name="s")

    @functools.partial(
        pl.kernel,
        mesh=mesh,
        out_type=jax.ShapeDtypeStruct((total, OUT_C), jnp.float32),
        scratch_types=[
            pltpu.VMEM((_SC_CHUNK,), jnp.int32),
            pltpu.VMEM((_SC_CHUNK, OUT_C), jnp.float32),
            pltpu.SemaphoreType.DMA,
        ],
    )
    def gather_k(table_hbm, idx_hbm, out_hbm, idx_v, rows_v, sem):
        wid = lax.axis_index("s") * info.num_cores + lax.axis_index("c")
        base = wid * per_w

        def body(j, carry):
            off = base + j * _SC_CHUNK
            pltpu.sync_copy(idx_hbm.at[pl.ds(off, _SC_CHUNK)], idx_v)
            pltpu.async_copy(table_hbm.at[idx_v], rows_v, sem).wait()
            pltpu.sync_copy(rows_v, out_hbm.at[pl.ds(off, _SC_CHUNK)])
            return carry

        lax.fori_loop(0, nchunk, body, 0)

    return gather_k(table, idx)


_LN_MB = 128


def _ln_body(g_ref, cx_ref, cy_ref, cz_ref, w3_ref, gam_ref, bet_ref, o_ref):
    cx = cx_ref[0]
    cy = cy_ref[0]
    cz = cz_ref[0]
    wx = w3_ref[0:1, :]
    wy = w3_ref[1:2, :]
    wz = w3_ref[2:3, :]
    projc = cx * wx + cy * wy + cz * wz
    gam = gam_ref[0:1, :]
    bet = bet_ref[0:1, :]

    acc = jnp.full((_LN_MB, OUT_C), -jnp.inf, dtype=jnp.float32)
    for k in range(NSAMPLE):
        z = g_ref[0, :, k] - projc
        mu = jnp.mean(z, axis=1, keepdims=True)
        zc = z - mu
        var = jnp.mean(zc * zc, axis=1, keepdims=True)
        y = zc * lax.rsqrt(var + EPS) * gam + bet
        acc = jnp.maximum(acc, y)
    o_ref[...] = jnp.maximum(acc, 0.0)


def _run_ln(g4, cxT, cyT, czT, w3, gamma2, beta2):
    nb = g4.shape[0]
    grid = (nb, M_PER // _LN_MB)
    cspec = pl.BlockSpec((1, _LN_MB, 1), lambda b, m: (b, m, 0))
    vspec = pl.BlockSpec((1, OUT_C), lambda b, m: (0, 0))
    return pl.pallas_call(
        _ln_body,
        grid=grid,
        in_specs=[
            pl.BlockSpec((1, _LN_MB, NSAMPLE, OUT_C), lambda b, m: (b, m, 0, 0)),
            cspec, cspec, cspec,
            pl.BlockSpec((3, OUT_C), lambda b, m: (0, 0)),
            vspec, vspec,
        ],
        out_specs=pl.BlockSpec((_LN_MB, OUT_C), lambda b, m: (b * (M_PER // _LN_MB) + m, 0)),
        out_shape=jax.ShapeDtypeStruct((nb * M_PER, OUT_C), jnp.float32),
        compiler_params=pltpu.CompilerParams(
            dimension_semantics=("parallel", "parallel")),
    )(g4, cxT, cyT, czT, w3, gamma2, beta2)


def kernel(p, x, o, W, gamma, beta):
    pb = p.reshape(B, N_PER, 3)
    pt = pb.reshape(B, _FS, _FL, 3).transpose(3, 0, 1, 2)
    ptb = pb.transpose(0, 2, 1)
    cx, cy, cz = (a.reshape(B, M_PER) for a in _run_fps(pt))
    cxT, cyT, czT = cx[..., None], cy[..., None], cz[..., None]

    knn = _run_knn(cxT, cyT, czT, ptb)

    w3 = W[:, :3].T
    wxt = W[:, 3:].T
    y_all = _run_proj(x, p, wxt, w3)

    idx_flat = knn.reshape(B * NSAMPLE * M_PER)
    per_b = NSAMPLE * M_PER
    gam2, bet2 = gamma[None, :], beta[None, :]
    grouped = [
        _run_sc_gather(y_all, idx_flat[b * per_b:(b + 1) * per_b])
        for b in range(B)
    ]
    x_out = jnp.concatenate([
        _run_ln(grouped[b].reshape(1, M_PER, NSAMPLE, OUT_C),
                cxT[b:b + 1], cyT[b:b + 1], czT[b:b + 1], w3, gam2, bet2)
        for b in range(B)
    ], axis=0)

    n_p = jnp.stack([cx, cy, cz], axis=-1).reshape(B * M_PER, 3)
    n_o = (jnp.arange(1, B + 1) * M_PER).astype(jnp.int32)
    return (n_p, x_out, n_o)


# speedup vs baseline: 14.6881x; 1.0018x over previous
import functools

import jax
import jax.numpy as jnp
from jax import lax
from jax.experimental import pallas as pl
from jax.experimental.pallas import tpu as pltpu
from jax.experimental.pallas import tpu_sc as plsc

B = 4
N_PER = 4096
STRIDE = 4
NSAMPLE = 16
IN_C = 256
OUT_C = 512
M_PER = N_PER // STRIDE
EPS = 1e-5


_FS = 8
_FL = N_PER // _FS
_MS = 8
_ML = M_PER // _MS


def _fps_body(pt_ref, cx_ref, cy_ref, cz_ref):
    px = pt_ref[0]
    py = pt_ref[1]
    pz = pt_ref[2]

    io = (lax.broadcasted_iota(jnp.int32, (B, _FS, _FL), 1) * _FL
          + lax.broadcasted_iota(jnp.int32, (B, _FS, _FL), 2))
    im = (lax.broadcasted_iota(jnp.int32, (B, _MS, _ML), 1) * _ML
          + lax.broadcasted_iota(jnp.int32, (B, _MS, _ML), 2))

    def _rmax(a):
        return jnp.max(jnp.max(a, axis=2, keepdims=True), axis=1,
                       keepdims=True)

    def _rmin(a):
        return jnp.min(jnp.min(a, axis=2, keepdims=True), axis=1,
                       keepdims=True)

    def _rsum(a):
        return jnp.sum(jnp.sum(a, axis=2, keepdims=True), axis=1,
                       keepdims=True)

    c0x = px[:, 0:1, 0:1]
    c0y = py[:, 0:1, 0:1]
    c0z = pz[:, 0:1, 0:1]
    dx = px - c0x
    dy = py - c0y
    dz = pz - c0z
    dist = (dx * dx + dy * dy) + dz * dz

    accx = jnp.where(im == 0, c0x, 0.0)
    accy = jnp.where(im == 0, c0y, 0.0)
    accz = jnp.where(im == 0, c0z, 0.0)

    def body(i, state):
        dist, accx, accy, accz = state
        mx = _rmax(dist)
        nxt = _rmin(jnp.where(dist == mx, io, N_PER))
        onehot = io == nxt
        cx = _rsum(jnp.where(onehot, px, 0.0))
        cy = _rsum(jnp.where(onehot, py, 0.0))
        cz = _rsum(jnp.where(onehot, pz, 0.0))
        dx = px - cx
        dy = py - cy
        dz = pz - cz
        d = (dx * dx + dy * dy) + dz * dz
        dist = jnp.minimum(dist, d)
        accx = jnp.where(im == i, cx, accx)
        accy = jnp.where(im == i, cy, accy)
        accz = jnp.where(im == i, cz, accz)
        return dist, accx, accy, accz

    def body2(j, state):
        state = body(2 + 2 * j, state)
        return body(3 + 2 * j, state)

    state = body(1, (dist, accx, accy, accz))
    _, accx, accy, accz = lax.fori_loop(0, (M_PER - 2) // 2, body2, state)
    cx_ref[...] = accx
    cy_ref[...] = accy
    cz_ref[...] = accz


def _run_fps(pt):
    out = jax.ShapeDtypeStruct((B, _MS, _ML), jnp.float32)
    return pl.pallas_call(
        _fps_body,
        out_shape=(out, out, out),
    )(pt)


_KNN_MB = 128


def _knn_body(cx_ref, cy_ref, cz_ref, pt_ref, knn_ref):
    b = pl.program_id(0)
    px = pt_ref[0, 0:1, :]
    py = pt_ref[0, 1:2, :]
    pz = pt_ref[0, 2:3, :]
    cx = cx_ref[0]
    cy = cy_ref[0]
    cz = cz_ref[0]

    dx = cx - px
    dy = cy - py
    dz = cz - pz
    d = (dx * dx + dy * dy) + dz * dz

    lane_n = lax.broadcasted_iota(jnp.int32, (_KNN_MB, N_PER), 1)
    base = b * N_PER
    inf = jnp.float32(jnp.inf)
    for k in range(NSAMPLE):
        am = jnp.argmin(d, axis=1).astype(jnp.int32)[:, None]
        knn_ref[0, :, k:k + 1] = am + base
        d = jnp.where(lane_n == am, inf, d)


def _run_knn(cxT, cyT, czT, ptb):
    grid = (B, M_PER // _KNN_MB)
    cspec = pl.BlockSpec((1, _KNN_MB, 1), lambda b, m: (b, m, 0))
    return pl.pallas_call(
        _knn_body,
        grid=grid,
        in_specs=[
            cspec, cspec, cspec,
            pl.BlockSpec((1, 3, N_PER), lambda b, m: (b, 0, 0)),
        ],
        out_specs=pl.BlockSpec((1, _KNN_MB, NSAMPLE), lambda b, m: (b, m, 0)),
        out_shape=jax.ShapeDtypeStruct((B, M_PER, NSAMPLE), jnp.int32),
        compiler_params=pltpu.CompilerParams(
            dimension_semantics=("parallel", "parallel")),
    )(cxT, cyT, czT, ptb)


_PROJ_RB = 1024


def _proj_body(x_ref, p_ref, wxt_ref, w3_ref, y_ref):
    y = jnp.dot(x_ref[...], wxt_ref[...], preferred_element_type=jnp.float32)
    y += p_ref[:, 0:1] * w3_ref[0:1, :]
    y += p_ref[:, 1:2] * w3_ref[1:2, :]
    y += p_ref[:, 2:3] * w3_ref[2:3, :]
    y_ref[...] = y


def _run_proj(x, p, wxt, w3):
    n = x.shape[0]
    grid = (n // _PROJ_RB,)
    return pl.pallas_call(
        _proj_body,
        grid=grid,
        in_specs=[
            pl.BlockSpec((_PROJ_RB, IN_C), lambda i: (i, 0)),
            pl.BlockSpec((_PROJ_RB, 3), lambda i: (i, 0)),
            pl.BlockSpec((IN_C, OUT_C), lambda i: (0, 0)),
            pl.BlockSpec((3, OUT_C), lambda i: (0, 0)),
        ],
        out_specs=pl.BlockSpec((_PROJ_RB, OUT_C), lambda i: (i, 0)),
        out_shape=jax.ShapeDtypeStruct((n, OUT_C), jnp.float32),
        compiler_params=pltpu.CompilerParams(
            dimension_semantics=("parallel",)),
    )(x, p, wxt, w3)


_SC_CHUNK = 128


def _run_sc_gather(table, idx):
    total = idx.shape[0]
    info = plsc.get_sparse_core_info()
    nw = info.num_cores * info.num_subcores
    per_w = total // nw
    nchunk = per_w // _SC_CHUNK
    mesh = plsc.VectorSubcoreMesh(core_axis_name="c", subcore_axis_name="s")

    @functools.partial(
        pl.kernel,
        mesh=mesh,
        out_type=jax.ShapeDtypeStruct((total, OUT_C), jnp.float32),
        scratch_types=[
            pltpu.VMEM((_SC_CHUNK,), jnp.int32),
            pltpu.VMEM((_SC_CHUNK, OUT_C), jnp.float32),
            pltpu.SemaphoreType.DMA,
        ],
    )
    def gather_k(table_hbm, idx_hbm, out_hbm, idx_v, rows_v, sem):
        wid = lax.axis_index("s") * info.num_cores + lax.axis_index("c")
        base = wid * per_w

        def body(j, carry):
            off = base + j * _SC_CHUNK
            pltpu.sync_copy(idx_hbm.at[pl.ds(off, _SC_CHUNK)], idx_v)
            pltpu.async_copy(table_hbm.at[idx_v], rows_v, sem).wait()
            pltpu.sync_copy(rows_v, out_hbm.at[pl.ds(off, _SC_CHUNK)])
            return carry

        lax.fori_loop(0, nchunk, body, 0)

    return gather_k(table, idx)


_LN_MB = 128


def _ln_body(g_ref, cx_ref, cy_ref, cz_ref, w3_ref, gam_ref, bet_ref, o_ref):
    cx = cx_ref[0]
    cy = cy_ref[0]
    cz = cz_ref[0]
    wx = w3_ref[0:1, :]
    wy = w3_ref[1:2, :]
    wz = w3_ref[2:3, :]
    projc = cx * wx + cy * wy + cz * wz
    gam = gam_ref[0:1, :]
    bet = bet_ref[0:1, :]

    acc = jnp.full((_LN_MB, OUT_C), -jnp.inf, dtype=jnp.float32)
    for k in range(NSAMPLE):
        z = g_ref[0, :, k] - projc
        mu = jnp.mean(z, axis=1, keepdims=True)
        zc = z - mu
        var = jnp.mean(zc * zc, axis=1, keepdims=True)
        y = zc * lax.rsqrt(var + EPS) * gam + bet
        acc = jnp.maximum(acc, y)
    o_ref[...] = jnp.maximum(acc, 0.0)


def _run_ln(g4, cxT, cyT, czT, w3, gamma2, beta2):
    nb = g4.shape[0]
    grid = (nb, M_PER // _LN_MB)
    cspec = pl.BlockSpec((1, _LN_MB, 1), lambda b, m: (b, m, 0))
    vspec = pl.BlockSpec((1, OUT_C), lambda b, m: (0, 0))
    return pl.pallas_call(
        _ln_body,
        grid=grid,
        in_specs=[
            pl.BlockSpec((1, _LN_MB, NSAMPLE, OUT_C), lambda b, m: (b, m, 0, 0)),
            cspec, cspec, cspec,
            pl.BlockSpec((3, OUT_C), lambda b, m: (0, 0)),
            vspec, vspec,
        ],
        out_specs=pl.BlockSpec((_LN_MB, OUT_C), lambda b, m: (b * (M_PER // _LN_MB) + m, 0)),
        out_shape=jax.ShapeDtypeStruct((nb * M_PER, OUT_C), jnp.float32),
        compiler_params=pltpu.CompilerParams(
            dimension_semantics=("parallel", "parallel")),
    )(g4, cxT, cyT, czT, w3, gamma2, beta2)


def kernel(p, x, o, W, gamma, beta):
    pb = p.reshape(B, N_PER, 3)
    pt = pb.reshape(B, _FS, _FL, 3).transpose(3, 0, 1, 2)
    ptb = pb.transpose(0, 2, 1)
    cx, cy, cz = (a.reshape(B, M_PER) for a in _run_fps(pt))
    cxT, cyT, czT = cx[..., None], cy[..., None], cz[..., None]

    knn = _run_knn(cxT, cyT, czT, ptb)

    w3 = W[:, :3].T
    wxt = W[:, 3:].T
    y_all = _run_proj(x, p, wxt, w3)

    idx_flat = knn.reshape(B * NSAMPLE * M_PER)
    per_b = NSAMPLE * M_PER
    gam2, bet2 = gamma[None, :], beta[None, :]
    grouped = [
        _run_sc_gather(y_all, idx_flat[b * per_b:(b + 1) * per_b])
        for b in range(B)
    ]
    x_out = jnp.concatenate([
        _run_ln(grouped[b].reshape(1, M_PER, NSAMPLE, OUT_C),
                cxT[b:b + 1], cyT[b:b + 1], czT[b:b + 1], w3, gam2, bet2)
        for b in range(B)
    ], axis=0)

    n_p = jnp.stack([cx, cy, cz], axis=-1).reshape(B * M_PER, 3)
    n_o = (jnp.arange(1, B + 1) * M_PER).astype(jnp.int32)
    return (n_p, x_out, n_o)
